# Initial kernel scaffold; baseline (speedup 1.0000x reference)
import jax
import jax.numpy as jnp
from jax.experimental import pallas as pl


def kernel(user_emb, item_emb, adj_row, adj_col, adj_val):
    raise NotImplementedError("write your pallas kernel here")


# speedup vs baseline: 10.6901x; 10.6901x over previous
import functools

import jax
import jax.numpy as jnp
from jax import lax
from jax.experimental import pallas as pl
from jax.experimental.pallas import tpu as pltpu
from jax.experimental.pallas import tpu_sc as plsc

N_USERS = 50000
N_ITEMS = 50000
N_NODES = N_USERS + N_ITEMS
D = 32
H = 16
NLAYERS = 3

NC = 2
NS = 16
LANES = 16

ROW_CHUNK = 128
ROW_CHUNKS_PER_TILE = 49
ROWS_PER_TILE = ROW_CHUNKS_PER_TILE * ROW_CHUNK
NP = NS * ROWS_PER_TILE

G = 128
SUPER = 8
CHUNKS_PER_TILE = 784
SUPERS_PER_TILE = CHUNKS_PER_TILE // SUPER
NE_PAD = NS * CHUNKS_PER_TILE * G
NCHUNKS = NE_PAD // G


def _gcn_body(h_hbm, col_hbm, row_hbm, val_hbm,
              out_hbm, t1_hbm, t2_hbm,
              acc, col_v, row_v, val_v, rows_v,
              gsem, ssem):
  c = lax.axis_index("c")
  s = lax.axis_index("s")

  row_base = s * ROWS_PER_TILE
  chunk_base = s * CHUNKS_PER_TILE

  b0 = rows_v.at[pl.ds(0 * ROW_CHUNK, ROW_CHUNK)]
  b1 = rows_v.at[pl.ds(1 * ROW_CHUNK, ROW_CHUNK)]
  b2 = rows_v.at[pl.ds(2 * ROW_CHUNK, ROW_CHUNK)]
  b3 = rows_v.at[pl.ds(3 * ROW_CHUNK, ROW_CHUNK)]
  zb = rows_v.at[pl.ds(4 * ROW_CHUNK, ROW_CHUNK)]

  def zero_zb():
    def zfill(r, _):
      zb[r] = jnp.zeros((LANES,), jnp.float32)
      return 0
    lax.fori_loop(0, ROW_CHUNK, zfill, 0)

  zero_zb()

  def zero_rows(k, _):
    pltpu.sync_copy(zb, acc.at[pl.ds(row_base + k * ROW_CHUNK, ROW_CHUNK)])
    return 0
  lax.fori_loop(0, ROW_CHUNKS_PER_TILE, zero_rows, 0)
  plsc.subcore_barrier()

  def edge_sweep(src_hbm):
    def super_body(sc_i, _):
      soff = chunk_base + sc_i * SUPER
      pltpu.sync_copy(col_hbm.at[pl.ds(soff, SUPER)], col_v)
      pltpu.sync_copy(row_hbm.at[pl.ds(soff, SUPER)], row_v)
      pltpu.sync_copy(val_hbm.at[pl.ds(soff * G, SUPER * G)], val_v)

      descs = []
      for j in range(SUPER):
        descs.append(pltpu.async_copy(
            src_hbm.at[col_v.at[j]],
            rows_v.at[pl.ds(j * G, G)], gsem))
      for d in descs:
        d.wait()

      def mul_body(e16, _):
        base = e16 * LANES
        vv = val_v[pl.ds(base, LANES)]
        for u in range(LANES):
          rows_v[base + u] = rows_v[base + u] * vv[u]
        return 0
      lax.fori_loop(0, (SUPER * G) // LANES, mul_body, 0)

      sdescs = []
      for j in range(SUPER):
        sdescs.append(pltpu.async_copy(
            rows_v.at[pl.ds(j * G, G)],
            acc.at[row_v.at[j]], ssem, add=True))
      for d in sdescs:
        d.wait()
      return 0
    lax.fori_loop(0, SUPERS_PER_TILE, super_body, 0)

  def drain(dst_hbm):
    zero_zb()
    def body(k, _):
      roff = row_base + k * ROW_CHUNK
      pltpu.sync_copy(acc.at[pl.ds(roff, ROW_CHUNK)], b0)
      pltpu.sync_copy(zb, acc.at[pl.ds(roff, ROW_CHUNK)])
      pltpu.sync_copy(b0, dst_hbm.at[c, pl.ds(roff, ROW_CHUNK)])
      return 0
    lax.fori_loop(0, ROW_CHUNKS_PER_TILE, body, 0)

  edge_sweep(h_hbm.at[c])
  plsc.subcore_barrier()
  drain(t1_hbm)
  plsc.subcore_barrier()

  edge_sweep(t1_hbm.at[c])
  plsc.subcore_barrier()
  drain(t2_hbm)
  plsc.subcore_barrier()

  edge_sweep(t2_hbm.at[c])
  plsc.subcore_barrier()

  def mean_body(k, _):
    roff = row_base + k * ROW_CHUNK
    pltpu.sync_copy(h_hbm.at[c, pl.ds(roff, ROW_CHUNK)], b0)
    pltpu.sync_copy(t1_hbm.at[c, pl.ds(roff, ROW_CHUNK)], b1)
    pltpu.sync_copy(t2_hbm.at[c, pl.ds(roff, ROW_CHUNK)], b2)
    pltpu.sync_copy(acc.at[pl.ds(roff, ROW_CHUNK)], b3)

    def add_body(r, _):
      b0[r] = (b0[r] + b1[r] + b2[r] + b3[r]) * 0.25
      return 0
    lax.fori_loop(0, ROW_CHUNK, add_body, 0)
    pltpu.sync_copy(b0, out_hbm.at[c, pl.ds(roff, ROW_CHUNK)])
    return 0
  lax.fori_loop(0, ROW_CHUNKS_PER_TILE, mean_body, 0)


@jax.jit
def _gcn(h, col2, row2, val2):
  f32 = jnp.float32
  mesh = plsc.VectorSubcoreMesh(core_axis_name="c", subcore_axis_name="s")
  run = pl.kernel(
      _gcn_body,
      out_type=[
          jax.ShapeDtypeStruct((NC, NP, H), f32),
          jax.ShapeDtypeStruct((NC, NP, H), f32),
          jax.ShapeDtypeStruct((NC, NP, H), f32),
      ],
      mesh=mesh,
      compiler_params=pltpu.CompilerParams(use_tc_tiling_on_sc=False),
      scratch_types=[
          pltpu.VMEM_SHARED((NP, H), f32),
          pltpu.VMEM((SUPER, G), jnp.int32),
          pltpu.VMEM((SUPER, G), jnp.int32),
          pltpu.VMEM((SUPER * G,), f32),
          pltpu.VMEM((SUPER * G, H), f32),
          pltpu.SemaphoreType.DMA,
          pltpu.SemaphoreType.DMA,
      ],
  )
  return run(h, col2, row2, val2)


def kernel(user_emb, item_emb, adj_row, adj_col, adj_val):
  all_emb = jnp.concatenate([user_emb, item_emb], axis=0)
  halves = jnp.stack([all_emb[:, :H], all_emb[:, H:]])
  h = jnp.zeros((NC, NP, H), jnp.float32).at[:, :N_NODES].set(halves)

  col = jnp.zeros((NE_PAD,), jnp.int32).at[:adj_col.shape[0]].set(
      adj_col.astype(jnp.int32)).reshape(NCHUNKS, G)
  row = jnp.zeros((NE_PAD,), jnp.int32).at[:adj_row.shape[0]].set(
      adj_row.astype(jnp.int32)).reshape(NCHUNKS, G)
  val = jnp.zeros((NE_PAD,), jnp.float32).at[:adj_val.shape[0]].set(adj_val)

  out, _, _ = _gcn(h, col, row, val)
  full = jnp.concatenate([out[0, :N_NODES], out[1, :N_NODES]], axis=1)
  return (full[:N_USERS], full[N_USERS:])


# speedup vs baseline: 15.4002x; 1.4406x over previous
import functools

import jax
import jax.numpy as jnp
from jax import lax
from jax.experimental import pallas as pl
from jax.experimental.pallas import tpu as pltpu
from jax.experimental.pallas import tpu_sc as plsc

N_USERS = 50000
N_ITEMS = 50000
N_NODES = N_USERS + N_ITEMS
D = 32
H = 16

NC = 2
NS = 16
LANES = 16

ROWS_PER_TILE = 6272
NP = NS * ROWS_PER_TILE
RC = 512
N_RC = 12
RC_TAIL = ROWS_PER_TILE - N_RC * RC
MC = 128
N_MC = ROWS_PER_TILE // MC

G = 128
SUPER = 4
SE = SUPER * G
CHUNKS_PER_TILE = 784
PAIRS_PER_TILE = CHUNKS_PER_TILE // (2 * SUPER)
NE_PAD = NS * CHUNKS_PER_TILE * G
NCHUNKS = NE_PAD // G


def _gcn_body(h_hbm, col_hbm, row_hbm, val_hbm,
              out_hbm, t1_hbm, t2_hbm,
              acc,
              colA, rowA, valA, rowsA,
              colB, rowB, valB, rowsB,
              zbuf,
              stA, stB, gsA, gsB, ssA, ssB, ws):
  c = lax.axis_index("c")
  s = lax.axis_index("s")

  row_base = s * ROWS_PER_TILE
  chunk_base = s * CHUNKS_PER_TILE

  def zfill(r, _):
    zbuf[r] = jnp.zeros((LANES,), jnp.float32)
    return 0
  lax.fori_loop(0, RC, zfill, 0)

  def zero_rows(k, _):
    pltpu.sync_copy(zbuf, acc.at[pl.ds(row_base + k * RC, RC)])
    return 0
  lax.fori_loop(0, N_RC, zero_rows, 0)
  pltpu.sync_copy(zbuf.at[pl.ds(0, RC_TAIL)],
                  acc.at[pl.ds(row_base + N_RC * RC, RC_TAIL)])
  plsc.subcore_barrier()

  def scale_rows(rows_ref, val_ref):
    @functools.partial(plsc.parallel_loop, 0, SE // LANES, unroll=2)
    def _mul(e16):
      base = e16 * LANES
      vv = val_ref[pl.ds(base, LANES)]
      for u in range(LANES):
        rows_ref[base + u] = rows_ref[base + u] * vv[u]

  def edge_sweep(src_hbm):
    def stage(soff, col_v, row_v, val_v, sem):
      d = [pltpu.async_copy(col_hbm.at[pl.ds(soff, SUPER)], col_v, sem),
           pltpu.async_copy(row_hbm.at[pl.ds(soff, SUPER)], row_v, sem),
           pltpu.async_copy(val_hbm.at[pl.ds(soff * G, SE)], val_v, sem)]
      return d

    def gathers(col_v, rows_v, sem):
      return [pltpu.async_copy(src_hbm.at[col_v.at[j]],
                               rows_v.at[pl.ds(j * G, G)], sem)
              for j in range(SUPER)]

    def scatters(row_v, rows_v, sem):
      return [pltpu.async_copy(rows_v.at[pl.ds(j * G, G)],
                               acc.at[row_v.at[j]], sem, add=True)
              for j in range(SUPER)]

    def pair_body(p, _):
      soffA = chunk_base + p * 2 * SUPER
      soffB = soffA + SUPER

      for d in stage(soffA, colA, rowA, valA, stA):
        d.wait()
      gA = gathers(colA, rowsA, gsA)
      for d in stage(soffB, colB, rowB, valB, stB):
        d.wait()
      gB = gathers(colB, rowsB, gsB)

      for d in gA:
        d.wait()
      scale_rows(rowsA, valA)
      sA = scatters(rowA, rowsA, ssA)

      for d in gB:
        d.wait()
      scale_rows(rowsB, valB)
      sB = scatters(rowB, rowsB, ssB)

      for d in sA:
        d.wait()
      for d in sB:
        d.wait()
      return 0
    lax.fori_loop(0, PAIRS_PER_TILE, pair_body, 0)

  def drain(dst_hbm):
    def body(k, _):
      roff = row_base + 2 * k * RC
      rd0 = pltpu.async_copy(acc.at[pl.ds(roff, RC)], rowsA, gsA)
      rd1 = pltpu.async_copy(acc.at[pl.ds(roff + RC, RC)], rowsB, gsB)
      rd0.wait()
      pltpu.sync_copy(zbuf, acc.at[pl.ds(roff, RC)])
      w0 = pltpu.async_copy(rowsA, dst_hbm.at[c, pl.ds(roff, RC)], ws)
      rd1.wait()
      pltpu.sync_copy(zbuf, acc.at[pl.ds(roff + RC, RC)])
      w1 = pltpu.async_copy(rowsB, dst_hbm.at[c, pl.ds(roff + RC, RC)], ws)
      w0.wait()
      w1.wait()
      return 0
    lax.fori_loop(0, N_RC // 2, body, 0)
    roff = row_base + N_RC * RC
    pltpu.sync_copy(acc.at[pl.ds(roff, RC_TAIL)], rowsA.at[pl.ds(0, RC_TAIL)])
    pltpu.sync_copy(zbuf.at[pl.ds(0, RC_TAIL)], acc.at[pl.ds(roff, RC_TAIL)])
    pltpu.sync_copy(rowsA.at[pl.ds(0, RC_TAIL)],
                    dst_hbm.at[c, pl.ds(roff, RC_TAIL)])

  edge_sweep(h_hbm.at[c])
  plsc.subcore_barrier()
  drain(t1_hbm)
  plsc.subcore_barrier()

  edge_sweep(t1_hbm.at[c])
  plsc.subcore_barrier()
  drain(t2_hbm)
  plsc.subcore_barrier()

  edge_sweep(t2_hbm.at[c])
  plsc.subcore_barrier()

  b0 = rowsA.at[pl.ds(0 * MC, MC)]
  b1 = rowsA.at[pl.ds(1 * MC, MC)]
  b2 = rowsA.at[pl.ds(2 * MC, MC)]
  b3 = rowsA.at[pl.ds(3 * MC, MC)]
  bo = rowsB.at[pl.ds(0, MC)]

  def mean_body(k, _):
    roff = row_base + k * MC
    r0 = pltpu.async_copy(h_hbm.at[c, pl.ds(roff, MC)], b0, stA)
    r1 = pltpu.async_copy(t1_hbm.at[c, pl.ds(roff, MC)], b1, stB)
    r2 = pltpu.async_copy(t2_hbm.at[c, pl.ds(roff, MC)], b2, gsA)
    r3 = pltpu.async_copy(acc.at[pl.ds(roff, MC)], b3, gsB)
    r0.wait(); r1.wait(); r2.wait(); r3.wait()

    def add_body(r, _):
      bo[r] = (b0[r] + b1[r] + b2[r] + b3[r]) * 0.25
      return 0
    lax.fori_loop(0, MC, add_body, 0)
    pltpu.sync_copy(bo, out_hbm.at[c, pl.ds(roff, MC)])
    return 0
  lax.fori_loop(0, N_MC, mean_body, 0)


@jax.jit
def _gcn(h, col2, row2, val2):
  f32 = jnp.float32
  mesh = plsc.VectorSubcoreMesh(core_axis_name="c", subcore_axis_name="s")
  run = pl.kernel(
      _gcn_body,
      out_type=[
          jax.ShapeDtypeStruct((NC, NP, H), f32),
          jax.ShapeDtypeStruct((NC, NP, H), f32),
          jax.ShapeDtypeStruct((NC, NP, H), f32),
      ],
      mesh=mesh,
      compiler_params=pltpu.CompilerParams(use_tc_tiling_on_sc=False),
      scratch_types=[
          pltpu.VMEM_SHARED((NP, H), f32),
          pltpu.VMEM((SUPER, G), jnp.int32),
          pltpu.VMEM((SUPER, G), jnp.int32),
          pltpu.VMEM((SE,), f32),
          pltpu.VMEM((SE, H), f32),
          pltpu.VMEM((SUPER, G), jnp.int32),
          pltpu.VMEM((SUPER, G), jnp.int32),
          pltpu.VMEM((SE,), f32),
          pltpu.VMEM((SE, H), f32),
          pltpu.VMEM((RC, H), f32),
          pltpu.SemaphoreType.DMA,
          pltpu.SemaphoreType.DMA,
          pltpu.SemaphoreType.DMA,
          pltpu.SemaphoreType.DMA,
          pltpu.SemaphoreType.DMA,
          pltpu.SemaphoreType.DMA,
          pltpu.SemaphoreType.DMA,
      ],
  )
  return run(h, col2, row2, val2)


def kernel(user_emb, item_emb, adj_row, adj_col, adj_val):
  all_emb = jnp.concatenate([user_emb, item_emb], axis=0)
  halves = jnp.stack([all_emb[:, :H], all_emb[:, H:]])
  h = jnp.zeros((NC, NP, H), jnp.float32).at[:, :N_NODES].set(halves)

  col = jnp.zeros((NE_PAD,), jnp.int32).at[:adj_col.shape[0]].set(
      adj_col.astype(jnp.int32)).reshape(NCHUNKS, G)
  row = jnp.zeros((NE_PAD,), jnp.int32).at[:adj_row.shape[0]].set(
      adj_row.astype(jnp.int32)).reshape(NCHUNKS, G)
  val = jnp.zeros((NE_PAD,), jnp.float32).at[:adj_val.shape[0]].set(adj_val)

  out, _, _ = _gcn(h, col, row, val)
  full = jnp.concatenate([out[0, :N_NODES], out[1, :N_NODES]], axis=1)
  return (full[:N_USERS], full[N_USERS:])


# speedup vs baseline: 19.4982x; 1.2661x over previous
import jax
import jax.numpy as jnp
from jax import lax
from jax.experimental import pallas as pl
from jax.experimental.pallas import tpu as pltpu
from jax.experimental.pallas import tpu_sc as plsc

N_USERS = 50000
N_ITEMS = 50000
N_NODES = N_USERS + N_ITEMS
D = 32
H = 16

NC = 2
NS = 16
LANES = 16

ROWS_PER_TILE = 6272
NP = NS * ROWS_PER_TILE
RC = 512
N_RC = 12
RC_TAIL = ROWS_PER_TILE - N_RC * RC

UPT = N_USERS // NS
MC = 125
N_MCH = UPT // MC

SE = 512
E_PER_TILE = 100000
PAIRS = 97
TAIL_OFF = PAIRS * 2 * SE
TB = 160


def _gcn_body(ue_hbm, ie_hbm, col_hbm, row_hbm, val_hbm,
              users_hbm, items_hbm, t0_hbm, t1_hbm, t2_hbm,
              acc,
              colA, rowA, valA, rowsA,
              colB, rowB, valB, rowsB,
              colT, rowT, valT, rowsT,
              zbuf,
              stA, stB, gsA, gsB, ssA, ssB, ws):
  c = lax.axis_index("c")
  s = lax.axis_index("s")

  row_base = s * ROWS_PER_TILE
  edge_base = s * E_PER_TILE
  u_base = s * UPT

  def zfill(r, _):
    zbuf[r] = jnp.zeros((LANES,), jnp.float32)
    return 0
  lax.fori_loop(0, RC, zfill, 0)

  def zero_rows(k, _):
    pltpu.sync_copy(zbuf, acc.at[pl.ds(row_base + k * RC, RC)])
    return 0
  lax.fori_loop(0, N_RC, zero_rows, 0)
  pltpu.sync_copy(zbuf.at[pl.ds(0, RC_TAIL)],
                  acc.at[pl.ds(row_base + N_RC * RC, RC_TAIL)])

  bA = rowsA.at[pl.ds(0, MC)]
  bB = rowsB.at[pl.ds(0, MC)]

  def build(src_hbm, dst_off):
    def body(k, _):
      r0 = u_base + 2 * k * MC
      rd0 = pltpu.async_copy(
          src_hbm.at[pl.ds(r0, MC), pl.ds(c * H, H)], bA, gsA)
      rd1 = pltpu.async_copy(
          src_hbm.at[pl.ds(r0 + MC, MC), pl.ds(c * H, H)], bB, gsB)
      rd0.wait()
      w0 = pltpu.async_copy(bA, t0_hbm.at[c, pl.ds(dst_off + r0, MC)], ws)
      rd1.wait()
      w1 = pltpu.async_copy(bB, t0_hbm.at[c, pl.ds(dst_off + r0 + MC, MC)], ws)
      w0.wait()
      w1.wait()
      return 0
    lax.fori_loop(0, N_MCH // 2, body, 0)
    r0 = u_base + (N_MCH - 1) * MC
    pltpu.sync_copy(src_hbm.at[pl.ds(r0, MC), pl.ds(c * H, H)], bA)
    pltpu.sync_copy(bA, t0_hbm.at[c, pl.ds(dst_off + r0, MC)])

  build(ue_hbm, 0)
  build(ie_hbm, N_USERS)
  plsc.subcore_barrier()

  def scale_rows(rows_ref, val_ref, n):
    def _mul(e16, carry):
      base = e16 * LANES
      vv = val_ref[pl.ds(base, LANES)]
      for u in range(LANES):
        rows_ref[base + u] = rows_ref[base + u] * vv[u]
      return carry
    lax.fori_loop(0, n // LANES, _mul, 0, unroll=2)

  def edge_sweep(src_hbm):
    def stage(eoff, col_v, row_v, val_v, sem, n):
      d = [pltpu.async_copy(col_hbm.at[pl.ds(eoff, n)], col_v, sem),
           pltpu.async_copy(row_hbm.at[pl.ds(eoff, n)], row_v, sem),
           pltpu.async_copy(val_hbm.at[pl.ds(eoff, n)], val_v, sem)]
      return d

    def pair_body(p, _):
      eoffA = edge_base + p * 2 * SE
      eoffB = eoffA + SE

      @pl.when(p != 0)
      def _():
        pltpu.make_async_copy(rowsA, acc.at[rowA], ssA).wait()
      for d in stage(eoffA, colA, rowA, valA, stA, SE):
        d.wait()
      gA = pltpu.async_copy(src_hbm.at[colA], rowsA, gsA)

      @pl.when(p != 0)
      def _():
        pltpu.make_async_copy(rowsB, acc.at[rowB], ssB).wait()
      for d in stage(eoffB, colB, rowB, valB, stB, SE):
        d.wait()
      gB = pltpu.async_copy(src_hbm.at[colB], rowsB, gsB)

      gA.wait()
      scale_rows(rowsA, valA, SE)
      pltpu.async_copy(rowsA, acc.at[rowA], ssA, add=True)

      gB.wait()
      scale_rows(rowsB, valB, SE)
      pltpu.async_copy(rowsB, acc.at[rowB], ssB, add=True)
      return 0
    lax.fori_loop(0, PAIRS, pair_body, 0)
    pltpu.make_async_copy(rowsA, acc.at[rowA], ssA).wait()
    pltpu.make_async_copy(rowsB, acc.at[rowB], ssB).wait()

    eoffA = edge_base + TAIL_OFF
    for d in stage(eoffA, colA, rowA, valA, stA, SE):
      d.wait()
    gA = pltpu.async_copy(src_hbm.at[colA], rowsA, gsA)
    for d in stage(eoffA + SE, colT, rowT, valT, stB, TB):
      d.wait()
    gT = pltpu.async_copy(src_hbm.at[colT], rowsT, gsB)
    gA.wait()
    scale_rows(rowsA, valA, SE)
    pltpu.async_copy(rowsA, acc.at[rowA], ssA, add=True)
    gT.wait()
    scale_rows(rowsT, valT, TB)
    pltpu.async_copy(rowsT, acc.at[rowT], ssB, add=True)
    pltpu.make_async_copy(rowsA, acc.at[rowA], ssA).wait()
    pltpu.make_async_copy(rowsT, acc.at[rowT], ssB).wait()

  def drain(dst_hbm):
    def body(k, _):
      roff = row_base + 2 * k * RC
      rd0 = pltpu.async_copy(acc.at[pl.ds(roff, RC)], rowsA, gsA)
      rd1 = pltpu.async_copy(acc.at[pl.ds(roff + RC, RC)], rowsB, gsB)
      rd0.wait()
      pltpu.sync_copy(zbuf, acc.at[pl.ds(roff, RC)])
      w0 = pltpu.async_copy(rowsA, dst_hbm.at[c, pl.ds(roff, RC)], ws)
      rd1.wait()
      pltpu.sync_copy(zbuf, acc.at[pl.ds(roff + RC, RC)])
      w1 = pltpu.async_copy(rowsB, dst_hbm.at[c, pl.ds(roff + RC, RC)], ws)
      w0.wait()
      w1.wait()
      return 0
    lax.fori_loop(0, N_RC // 2, body, 0)
    roff = row_base + N_RC * RC
    pltpu.sync_copy(acc.at[pl.ds(roff, RC_TAIL)], rowsA.at[pl.ds(0, RC_TAIL)])
    pltpu.sync_copy(zbuf.at[pl.ds(0, RC_TAIL)], acc.at[pl.ds(roff, RC_TAIL)])
    pltpu.sync_copy(rowsA.at[pl.ds(0, RC_TAIL)],
                    dst_hbm.at[c, pl.ds(roff, RC_TAIL)])

  edge_sweep(t0_hbm.at[c])
  plsc.subcore_barrier()
  drain(t1_hbm)
  plsc.subcore_barrier()

  edge_sweep(t1_hbm.at[c])
  plsc.subcore_barrier()
  drain(t2_hbm)
  plsc.subcore_barrier()

  edge_sweep(t2_hbm.at[c])
  plsc.subcore_barrier()

  m0 = rowsA.at[pl.ds(0, MC)]
  m1 = rowsA.at[pl.ds(128, MC)]
  m2 = rowsA.at[pl.ds(256, MC)]
  m3 = rowsA.at[pl.ds(384, MC)]
  mo = rowsB.at[pl.ds(0, MC)]

  def mean(out_hbm, src_off):
    def body(k, _):
      r0 = u_base + k * MC
      roff = src_off + r0
      r0d = pltpu.async_copy(t0_hbm.at[c, pl.ds(roff, MC)], m0, stA)
      r1d = pltpu.async_copy(t1_hbm.at[c, pl.ds(roff, MC)], m1, stB)
      r2d = pltpu.async_copy(t2_hbm.at[c, pl.ds(roff, MC)], m2, gsA)
      r3d = pltpu.async_copy(acc.at[pl.ds(roff, MC)], m3, gsB)
      r0d.wait(); r1d.wait(); r2d.wait(); r3d.wait()

      def add_body(r, carry):
        mo[r] = (m0[r] + m1[r] + m2[r] + m3[r]) * 0.25
        return carry
      lax.fori_loop(0, MC, add_body, 0, unroll=2)
      pltpu.sync_copy(mo, out_hbm.at[pl.ds(r0, MC), pl.ds(c * H, H)])
      return 0
    lax.fori_loop(0, N_MCH, body, 0)

  mean(users_hbm, 0)
  mean(items_hbm, N_USERS)


@jax.jit
def _gcn(ue, ie, col, row, val):
  f32 = jnp.float32
  mesh = plsc.VectorSubcoreMesh(core_axis_name="c", subcore_axis_name="s")
  run = pl.kernel(
      _gcn_body,
      out_type=[
          jax.ShapeDtypeStruct((N_USERS, D), f32),
          jax.ShapeDtypeStruct((N_ITEMS, D), f32),
          jax.ShapeDtypeStruct((NC, NP, H), f32),
          jax.ShapeDtypeStruct((NC, NP, H), f32),
          jax.ShapeDtypeStruct((NC, NP, H), f32),
      ],
      mesh=mesh,
      compiler_params=pltpu.CompilerParams(use_tc_tiling_on_sc=False),
      scratch_types=[
          pltpu.VMEM_SHARED((NP, H), f32),
          pltpu.VMEM((SE,), jnp.int32),
          pltpu.VMEM((SE,), jnp.int32),
          pltpu.VMEM((SE,), f32),
          pltpu.VMEM((SE, H), f32),
          pltpu.VMEM((SE,), jnp.int32),
          pltpu.VMEM((SE,), jnp.int32),
          pltpu.VMEM((SE,), f32),
          pltpu.VMEM((SE, H), f32),
          pltpu.VMEM((TB,), jnp.int32),
          pltpu.VMEM((TB,), jnp.int32),
          pltpu.VMEM((TB,), f32),
          pltpu.VMEM((TB, H), f32),
          pltpu.VMEM((RC, H), f32),
          pltpu.SemaphoreType.DMA,
          pltpu.SemaphoreType.DMA,
          pltpu.SemaphoreType.DMA,
          pltpu.SemaphoreType.DMA,
          pltpu.SemaphoreType.DMA,
          pltpu.SemaphoreType.DMA,
          pltpu.SemaphoreType.DMA,
      ],
  )
  return run(ue, ie, col, row, val)


def kernel(user_emb, item_emb, adj_row, adj_col, adj_val):
  users, items, _, _, _ = _gcn(user_emb, item_emb,
                               adj_col.astype(jnp.int32),
                               adj_row.astype(jnp.int32),
                               adj_val)
  return (users, items)
